# Initial kernel scaffold; baseline (speedup 1.0000x reference)
import jax
import jax.numpy as jnp
from jax.experimental import pallas as pl


def kernel(X, table):
    raise NotImplementedError("write your pallas kernel here")


# speedup vs baseline: 1.8705x; 1.8705x over previous
import numpy as np
import jax
import jax.numpy as jnp
from jax import lax
from jax.experimental import pallas as pl
from jax.experimental.pallas import tpu as pltpu
from jax.experimental.pallas import tpu_sc as plsc

D_MODEL = 128
MAX_LEN = 512
CHUNK = 128


def _positional_encoding(d_model, max_len):
    pos = np.arange(0, max_len).reshape(-1, 1) / np.power(
        10000.0, np.arange(0, d_model, 2) / d_model)
    pe = np.zeros((max_len, d_model), dtype=np.float32)
    pe[:, 0::2] = np.sin(pos)
    pe[:, 1::2] = np.cos(pos)
    return pe


def _build(B, S, V):
    NC, NS = 2, 16
    NW = NC * NS
    total = B * S
    assert total % (NW * CHUNK) == 0
    per_w = total // NW
    n_chunks = per_w // CHUNK
    assert CHUNK <= S

    mesh = plsc.VectorSubcoreMesh(core_axis_name="c", subcore_axis_name="s")

    @pl.kernel(
        out_type=jax.ShapeDtypeStruct((total, D_MODEL), jnp.float32),
        mesh=mesh,
        scratch_types=[
            pltpu.VMEM((1, n_chunks, CHUNK), jnp.int32),
            pltpu.VMEM((2 * S, D_MODEL), jnp.float32),
            pltpu.VMEM((CHUNK, D_MODEL), jnp.float32),
            pltpu.SemaphoreType.DMA,
        ],
    )
    def k(table_hbm, idx_hbm, pe_hbm, out_hbm, idx_v, pe_v, rows_v, sem):
        cid = lax.axis_index("c")
        sid = lax.axis_index("s")
        wid = sid * NC + cid
        pltpu.sync_copy(idx_hbm.at[pl.ds(wid, 1)], idx_v)
        pltpu.sync_copy(pe_hbm, pe_v)

        @pl.loop(0, n_chunks)
        def _chunk(j):
            pltpu.async_copy(table_hbm.at[idx_v.at[0, j]], rows_v, sem).wait()
            off = lax.rem(j * CHUNK, S)

            @pl.loop(0, CHUNK)
            def _row(r):
                for c in range(D_MODEL // 16):
                    s = pl.ds(c * 16, 16)
                    rows_v[r, s] = rows_v[r, s] + pe_v[off + r, s]

            pltpu.sync_copy(
                rows_v, out_hbm.at[pl.ds(wid * per_w + j * CHUNK, CHUNK)])

    return k


def kernel(X, table):
    B, S = X.shape
    V, D = table.shape
    assert D == D_MODEL
    pe_np = _positional_encoding(D_MODEL, MAX_LEN)[:S]
    pe = jnp.asarray(np.concatenate([pe_np, pe_np], axis=0))
    NW = 32
    idx3d = X.astype(jnp.int32).reshape(NW, -1, CHUNK)
    k = _build(B, S, V)
    out = k(table, idx3d, pe)
    return out.reshape(B, S, D)


# speedup vs baseline: 2.2541x; 1.2051x over previous
import numpy as np
import jax
import jax.numpy as jnp
from jax import lax
from jax.experimental import pallas as pl
from jax.experimental.pallas import tpu as pltpu
from jax.experimental.pallas import tpu_sc as plsc

D_MODEL = 128
MAX_LEN = 512
CHUNK = 128


def _positional_encoding(d_model, max_len):
    pos = np.arange(0, max_len).reshape(-1, 1) / np.power(
        10000.0, np.arange(0, d_model, 2) / d_model)
    pe = np.zeros((max_len, d_model), dtype=np.float32)
    pe[:, 0::2] = np.sin(pos)
    pe[:, 1::2] = np.cos(pos)
    return pe


def _build(B, S, V):
    NC, NS = 2, 16
    NW = NC * NS
    total = B * S
    assert total % (NW * CHUNK) == 0
    per_w = total // NW
    n_chunks = per_w // CHUNK
    assert CHUNK <= S

    mesh = plsc.VectorSubcoreMesh(core_axis_name="c", subcore_axis_name="s")

    n_pairs = n_chunks // 2
    assert n_chunks % 2 == 0

    @pl.kernel(
        out_type=jax.ShapeDtypeStruct((total, D_MODEL), jnp.float32),
        mesh=mesh,
        scratch_types=[
            pltpu.VMEM((1, n_chunks, CHUNK), jnp.int32),
            pltpu.VMEM((2 * S, D_MODEL), jnp.float32),
            pltpu.VMEM((CHUNK, D_MODEL), jnp.float32),
            pltpu.VMEM((CHUNK, D_MODEL), jnp.float32),
            pltpu.SemaphoreType.DMA,
            pltpu.SemaphoreType.DMA,
            pltpu.SemaphoreType.DMA,
            pltpu.SemaphoreType.DMA,
        ],
    )
    def k(table_hbm, idx_hbm, pe_hbm, out_hbm,
          idx_v, pe_v, rows0, rows1, g0, g1, w0, w1):
        cid = lax.axis_index("c")
        sid = lax.axis_index("s")
        wid = sid * NC + cid
        bufs = (rows0, rows1)
        gsems = (g0, g1)
        wsems = (w0, w1)

        pltpu.sync_copy(idx_hbm.at[pl.ds(wid, 1)], idx_v)

        def gather_start(j, b):
            pltpu.async_copy(table_hbm.at[idx_v.at[0, j]], bufs[b], gsems[b])

        def gather_wait(b):
            pltpu.make_async_copy(
                table_hbm.at[pl.ds(0, CHUNK)], bufs[b], gsems[b]).wait()

        def write_start(j, b):
            pltpu.async_copy(
                bufs[b], out_hbm.at[pl.ds(wid * per_w + j * CHUNK, CHUNK)],
                wsems[b])

        def write_wait(b):
            pltpu.make_async_copy(
                bufs[b], out_hbm.at[pl.ds(0, CHUNK)], wsems[b]).wait()

        def pe_add(j, b):
            off = lax.rem(j * CHUNK, S)
            rows = bufs[b]

            @pl.loop(0, CHUNK)
            def _row(r):
                for c in range(D_MODEL // 16):
                    s = pl.ds(c * 16, 16)
                    rows[r, s] = rows[r, s] + pe_v[off + r, s]

        gather_start(0, 0)
        gather_start(1, 1)
        pltpu.sync_copy(pe_hbm, pe_v)

        @pl.loop(0, n_pairs)
        def _pair(kk):
            j0 = kk * 2
            for b in range(2):
                j = j0 + b
                gather_wait(b)
                pe_add(j, b)
                write_start(j, b)

            @pl.when(kk < n_pairs - 1)
            def _refill():
                for b in range(2):
                    write_wait(b)
                    gather_start(j0 + 2 + b, b)

        write_wait(0)
        write_wait(1)

    return k


def kernel(X, table):
    B, S = X.shape
    V, D = table.shape
    assert D == D_MODEL
    pe_np = _positional_encoding(D_MODEL, MAX_LEN)[:S]
    pe = jnp.asarray(np.concatenate([pe_np, pe_np], axis=0))
    NW = 32
    idx3d = X.astype(jnp.int32).reshape(NW, -1, CHUNK)
    k = _build(B, S, V)
    out = k(table, idx3d, pe)
    return out.reshape(B, S, D)


# speedup vs baseline: 5.6430x; 2.5035x over previous
import numpy as np
import jax
import jax.numpy as jnp
from jax import lax
from jax.experimental import pallas as pl
from jax.experimental.pallas import tpu as pltpu
from jax.experimental.pallas import tpu_sc as plsc

D_MODEL = 128
MAX_LEN = 512
CHUNK = 128


def _positional_encoding(d_model, max_len):
    pos = np.arange(0, max_len).reshape(-1, 1) / np.power(
        10000.0, np.arange(0, d_model, 2) / d_model)
    pe = np.zeros((max_len, d_model), dtype=np.float32)
    pe[:, 0::2] = np.sin(pos)
    pe[:, 1::2] = np.cos(pos)
    return pe


def _build(B, S, V):
    NC, NS = 2, 16
    NW = NC * NS
    total = B * S
    assert total % (NW * CHUNK) == 0
    per_w = total // NW
    n_chunks = per_w // CHUNK
    assert CHUNK <= S

    mesh = plsc.VectorSubcoreMesh(core_axis_name="c", subcore_axis_name="s")

    n_pairs = n_chunks // 2
    assert n_chunks % 2 == 0

    @pl.kernel(
        out_type=jax.ShapeDtypeStruct((total, D_MODEL), jnp.float32),
        mesh=mesh,
        scratch_types=[
            pltpu.VMEM((1, n_chunks, CHUNK), jnp.int32),
            pltpu.VMEM((2 * S, D_MODEL), jnp.float32),
            pltpu.VMEM((CHUNK, D_MODEL), jnp.float32),
            pltpu.VMEM((CHUNK, D_MODEL), jnp.float32),
            pltpu.SemaphoreType.DMA,
            pltpu.SemaphoreType.DMA,
            pltpu.SemaphoreType.DMA,
            pltpu.SemaphoreType.DMA,
        ],
    )
    def k(table_hbm, idx_hbm, pe_hbm, out_hbm,
          idx_v, pe_v, rows0, rows1, g0, g1, w0, w1):
        cid = lax.axis_index("c")
        sid = lax.axis_index("s")
        wid = sid * NC + cid
        bufs = (rows0, rows1)
        gsems = (g0, g1)
        wsems = (w0, w1)

        pltpu.sync_copy(idx_hbm.at[pl.ds(wid, 1)], idx_v)

        def gather_start(j, b):
            pltpu.async_copy(table_hbm.at[idx_v.at[0, j]], bufs[b], gsems[b])

        def gather_wait(b):
            pltpu.make_async_copy(
                table_hbm.at[pl.ds(0, CHUNK)], bufs[b], gsems[b]).wait()

        def write_start(j, b):
            pltpu.async_copy(
                bufs[b], out_hbm.at[pl.ds(wid * per_w + j * CHUNK, CHUNK)],
                wsems[b])

        def write_wait(b):
            pltpu.make_async_copy(
                bufs[b], out_hbm.at[pl.ds(0, CHUNK)], wsems[b]).wait()

        def pe_add(j, b):
            off = lax.rem(j * CHUNK, S)
            rows = bufs[b]

            @plsc.parallel_loop(0, CHUNK, unroll=4)
            def _row(r):
                for c in range(D_MODEL // 16):
                    s = pl.ds(c * 16, 16)
                    rows[r, s] = rows[r, s] + pe_v[off + r, s]

        gather_start(0, 0)
        gather_start(1, 1)
        pltpu.sync_copy(pe_hbm, pe_v)

        @pl.loop(0, n_pairs)
        def _pair(kk):
            j0 = kk * 2
            for b in range(2):
                j = j0 + b
                gather_wait(b)
                pe_add(j, b)
                write_start(j, b)

            @pl.when(kk < n_pairs - 1)
            def _refill():
                for b in range(2):
                    write_wait(b)
                    gather_start(j0 + 2 + b, b)

        write_wait(0)
        write_wait(1)

    return k


def kernel(X, table):
    B, S = X.shape
    V, D = table.shape
    assert D == D_MODEL
    pe_np = _positional_encoding(D_MODEL, MAX_LEN)[:S]
    pe = jnp.asarray(np.concatenate([pe_np, pe_np], axis=0))
    NW = 32
    idx3d = X.astype(jnp.int32).reshape(NW, -1, CHUNK)
    k = _build(B, S, V)
    out = k(table, idx3d, pe)
    return out.reshape(B, S, D)


# speedup vs baseline: 6.2239x; 1.1029x over previous
import numpy as np
import jax
import jax.numpy as jnp
from jax import lax
from jax.experimental import pallas as pl
from jax.experimental.pallas import tpu as pltpu
from jax.experimental.pallas import tpu_sc as plsc

D_MODEL = 128
MAX_LEN = 512
CHUNK = 128


def _positional_encoding(d_model, max_len):
    pos = np.arange(0, max_len).reshape(-1, 1) / np.power(
        10000.0, np.arange(0, d_model, 2) / d_model)
    pe = np.zeros((max_len, d_model), dtype=np.float32)
    pe[:, 0::2] = np.sin(pos)
    pe[:, 1::2] = np.cos(pos)
    return pe


def _build(B, S, V):
    NC, NS = 2, 16
    NW = NC * NS
    total = B * S
    assert total % (NW * CHUNK) == 0
    per_w = total // NW
    n_chunks = per_w // CHUNK
    assert CHUNK <= S
    pe_rows = S + CHUNK

    NBUF = 5
    n_groups = n_chunks // NBUF
    assert n_chunks % NBUF == 0

    mesh = plsc.VectorSubcoreMesh(core_axis_name="c", subcore_axis_name="s")

    @pl.kernel(
        out_type=jax.ShapeDtypeStruct((total, D_MODEL), jnp.float32),
        mesh=mesh,
        scratch_types=[
            pltpu.VMEM((1, n_chunks, CHUNK), jnp.int32),
            pltpu.VMEM((pe_rows, D_MODEL), jnp.float32),
        ] + [pltpu.VMEM((CHUNK, D_MODEL), jnp.float32)] * NBUF
          + [pltpu.SemaphoreType.DMA] * (2 * NBUF),
    )
    def k(table_hbm, idx_hbm, pe_hbm, out_hbm, idx_v, pe_v, *bufs_sems):
        bufs = bufs_sems[:NBUF]
        gsems = bufs_sems[NBUF:2 * NBUF]
        wsems = bufs_sems[2 * NBUF:]
        cid = lax.axis_index("c")
        sid = lax.axis_index("s")
        wid = sid * NC + cid

        pltpu.sync_copy(idx_hbm.at[pl.ds(wid, 1)], idx_v)

        def gather_start(j, b):
            pltpu.async_copy(table_hbm.at[idx_v.at[0, j]], bufs[b], gsems[b])

        def gather_wait(b):
            pltpu.make_async_copy(
                table_hbm.at[pl.ds(0, CHUNK)], bufs[b], gsems[b]).wait()

        def write_start(j, b):
            pltpu.async_copy(
                bufs[b], out_hbm.at[pl.ds(wid * per_w + j * CHUNK, CHUNK)],
                wsems[b])

        def write_wait(b):
            pltpu.make_async_copy(
                bufs[b], out_hbm.at[pl.ds(0, CHUNK)], wsems[b]).wait()

        def pe_add(j, b):
            off = lax.rem(j * CHUNK, S)
            rows = bufs[b]

            @plsc.parallel_loop(0, CHUNK, unroll=4)
            def _row(r):
                for c in range(D_MODEL // 16):
                    s = pl.ds(c * 16, 16)
                    rows[r, s] = rows[r, s] + pe_v[off + r, s]

        for b in range(NBUF):
            gather_start(b, b)
        pltpu.sync_copy(pe_hbm, pe_v)

        @pl.loop(0, n_groups)
        def _group(kk):
            j0 = kk * NBUF
            for b in range(NBUF):
                j = j0 + b
                gather_wait(b)
                pe_add(j, b)
                write_start(j, b)

                @pl.when(kk < n_groups - 1)
                def _refill():
                    write_wait(b)
                    gather_start(j + NBUF, b)

        for b in range(NBUF):
            write_wait(b)

    return k


def kernel(X, table):
    B, S = X.shape
    V, D = table.shape
    assert D == D_MODEL
    pe_np = _positional_encoding(D_MODEL, MAX_LEN)[:S]
    pe = jnp.asarray(np.concatenate([pe_np, pe_np[:CHUNK]], axis=0))
    NW = 32
    idx3d = X.astype(jnp.int32).reshape(NW, -1, CHUNK)
    k = _build(B, S, V)
    out = k(table, idx3d, pe)
    return out.reshape(B, S, D)
